# Initial kernel scaffold; baseline (speedup 1.0000x reference)
import jax
import jax.numpy as jnp
from jax.experimental import pallas as pl


def kernel(td, norm_factors):
    raise NotImplementedError("write your pallas kernel here")


# speedup vs baseline: 2.6820x; 2.6820x over previous
import functools

import jax
import jax.numpy as jnp
from jax import lax
from jax.experimental import pallas as pl
from jax.experimental.pallas import tpu as pltpu
from jax.experimental.pallas import tpu_sc as plsc

_B, _T, _C = 16384, 200, 3
_NFLOAT = _B * _T * _C
_NW = 32
_PER_W = _NFLOAT // _NW
_CHUNK = 30720
_NCHUNK = _PER_W // _CHUNK
_TRIPS = _CHUNK // 48


def _sc_body(td_hbm, tab_hbm, out_hbm, tab_v, buf):
    cid = lax.axis_index("c")
    sid = lax.axis_index("s")
    wid = sid * 2 + cid

    pltpu.sync_copy(tab_hbm, tab_v)

    def inv_body(k, _):
        sl = pl.ds(k * 16, 16)
        tab_v[sl] = 1.0 / tab_v[sl]
        return 0

    lax.fori_loop(0, 16, inv_body, 0)

    i3 = lax.iota(jnp.int32, 16) * 3

    def chunk_body(c, _):
        off = wid * _PER_W + c * _CHUNK
        pltpu.sync_copy(td_hbm.at[pl.ds(off, _CHUNK)], buf)

        def trip_body(i, _):
            pi = i3 + (i * 48 + 1)
            pv = pi + 1
            idx = plsc.load_gather(buf, [pi]).astype(jnp.int32)
            nf = plsc.load_gather(tab_v, [idx])
            val = plsc.load_gather(buf, [pv])
            plsc.store_scatter(buf, [pv], val * nf)
            return 0

        lax.fori_loop(0, _TRIPS, trip_body, 0)
        pltpu.sync_copy(buf, out_hbm.at[pl.ds(off, _CHUNK)])
        return 0

    lax.fori_loop(0, _NCHUNK, chunk_body, 0)


@jax.jit
def kernel(td, norm_factors):
    mesh = plsc.VectorSubcoreMesh(core_axis_name="c", subcore_axis_name="s")
    flat = td.reshape(_NFLOAT)
    out = pl.kernel(
        _sc_body,
        out_type=jax.ShapeDtypeStruct((_NFLOAT,), jnp.float32),
        mesh=mesh,
        scratch_types=[
            pltpu.VMEM((256,), jnp.float32),
            pltpu.VMEM((_CHUNK,), jnp.float32),
        ],
        compiler_params=pltpu.CompilerParams(needs_layout_passes=False),
    )(flat, norm_factors)
    return out.reshape(_B, _T, _C)


# speedup vs baseline: 11.2637x; 4.1998x over previous
import jax
import jax.numpy as jnp
from jax import lax
from jax.experimental import pallas as pl
from jax.experimental.pallas import tpu as pltpu
from jax.experimental.pallas import tpu_sc as plsc

_B, _T, _C = 16384, 200, 3
_NW = 32
_ROWS_W = _B // _NW
_RB = 4
_NCHUNK = _ROWS_W // _RB
_GROUPS = _RB * _T // 16


def _sc_body(td_hbm, tab_hbm, out_hbm, tab_v, buf):
    cid = lax.axis_index("c")
    sid = lax.axis_index("s")
    wid = sid * 2 + cid

    pltpu.sync_copy(tab_hbm, tab_v)

    def inv_body(k, _):
        sl = pl.ds(k * 16, 16)
        tab_v[sl] = 1.0 / tab_v[sl]
        return 0

    lax.fori_loop(0, 16, inv_body, 0)

    lanes = lax.iota(jnp.int32, 16)
    ones = lanes * 0 + 1
    twos = lanes * 0 + 2

    def chunk_body(c, _):
        r0 = wid * _ROWS_W + c * _RB
        pltpu.sync_copy(td_hbm.at[pl.ds(r0, _RB)], buf)

        def g_body(g, bt):
            b, t = bt
            idx = plsc.load_gather(buf, [b, t, ones]).astype(jnp.int32)
            nf = plsc.load_gather(tab_v, [idx])
            val = plsc.load_gather(buf, [b, t, twos])
            plsc.store_scatter(buf, [b, t, twos], val * nf)
            t2 = t + 16
            wrap = t2 >= _T
            return (jnp.where(wrap, b + 1, b), jnp.where(wrap, t2 - _T, t2))

        lax.fori_loop(0, _GROUPS, g_body, (lanes * 0, lanes))
        pltpu.sync_copy(buf, out_hbm.at[pl.ds(r0, _RB)])
        return 0

    lax.fori_loop(0, _NCHUNK, chunk_body, 0)


@jax.jit
def kernel(td, norm_factors):
    mesh = plsc.VectorSubcoreMesh(core_axis_name="c", subcore_axis_name="s")
    return pl.kernel(
        _sc_body,
        out_type=jax.ShapeDtypeStruct((_B, _T, _C), jnp.float32),
        mesh=mesh,
        scratch_types=[
            pltpu.VMEM((256,), jnp.float32),
            pltpu.VMEM((_RB, _T, _C), jnp.float32),
        ],
        compiler_params=pltpu.CompilerParams(
            needs_layout_passes=False,
        ),
    )(td, norm_factors)


# speedup vs baseline: 12.2314x; 1.0859x over previous
import jax
import jax.numpy as jnp
from jax import lax
from jax.experimental import pallas as pl
from jax.experimental.pallas import tpu as pltpu
from jax.experimental.pallas import tpu_sc as plsc

_B, _T, _C = 16384, 200, 3
_NW = 32
_ROWS_W = _B // _NW
_NBUF = 3
_FULLG = _T // 16
_TAIL = _T - _FULLG * 16


def _sc_body(td_hbm, tab_hbm, out_hbm, tab_v, buf, sin, sout):
    cid = lax.axis_index("c")
    sid = lax.axis_index("s")
    wid = sid * 2 + cid
    row0 = wid * _ROWS_W

    pltpu.sync_copy(tab_hbm, tab_v)

    def inv_body(k, _):
        sl = pl.ds(k * 16, 16)
        tab_v[sl] = 1.0 / tab_v[sl]
        return 0

    lax.fori_loop(0, 16, inv_body, 0)

    lanes = lax.iota(jnp.int32, 16)
    ones = lanes * 0 + 1
    twos = lanes * 0 + 2
    tailmask = lanes < _TAIL

    def start_in(p, c):
        pltpu.make_async_copy(
            td_hbm.at[row0 + c], buf.at[p], sin.at[p]
        ).start()

    def wait_in(p, c):
        pltpu.make_async_copy(
            td_hbm.at[row0 + c], buf.at[p], sin.at[p]
        ).wait()

    def start_out(p, c):
        pltpu.make_async_copy(
            buf.at[p], out_hbm.at[row0 + c], sout.at[p]
        ).start()

    def wait_out(p, c):
        pltpu.make_async_copy(
            buf.at[p], out_hbm.at[row0 + c], sout.at[p]
        ).wait()

    start_in(0, 0)
    start_in(1, 1)

    def chunk_body(c, _):
        p = c % _NBUF
        pvec = lanes * 0 + p
        wait_in(p, c)

        def g_body(g, _):
            t = lanes + g * 16
            idx = plsc.load_gather(buf, [pvec, t, ones]).astype(jnp.int32)
            nf = plsc.load_gather(tab_v, [idx])
            val = plsc.load_gather(buf, [pvec, t, twos])
            plsc.store_scatter(buf, [pvec, t, twos], val * nf)
            return 0

        lax.fori_loop(0, _FULLG, g_body, 0)
        t = lanes + _FULLG * 16
        idx = plsc.load_gather(buf, [pvec, t, ones], mask=tailmask)
        idx = idx.astype(jnp.int32)
        nf = plsc.load_gather(tab_v, [idx])
        val = plsc.load_gather(buf, [pvec, t, twos], mask=tailmask)
        plsc.store_scatter(buf, [pvec, t, twos], val * nf, mask=tailmask)

        start_out(p, c)
        q = (c + 2) % _NBUF

        @pl.when(c >= 1)
        def _():
            wait_out(q, c - 1)

        @pl.when(c + 2 < _ROWS_W)
        def _():
            start_in(q, c + 2)

        return 0

    lax.fori_loop(0, _ROWS_W, chunk_body, 0)
    wait_out((_ROWS_W - 1) % _NBUF, _ROWS_W - 1)


@jax.jit
def kernel(td, norm_factors):
    mesh = plsc.VectorSubcoreMesh(core_axis_name="c", subcore_axis_name="s")
    return pl.kernel(
        _sc_body,
        out_type=jax.ShapeDtypeStruct((_B, _T, _C), jnp.float32),
        mesh=mesh,
        scratch_types=[
            pltpu.VMEM((256,), jnp.float32),
            pltpu.VMEM((_NBUF, _T, _C), jnp.float32),
            pltpu.SemaphoreType.DMA((_NBUF,)),
            pltpu.SemaphoreType.DMA((_NBUF,)),
        ],
        compiler_params=pltpu.CompilerParams(
            needs_layout_passes=False,
        ),
    )(td, norm_factors)
